# baseline (speedup 1.0000x reference)
import jax
import jax.numpy as jnp
from jax.experimental import pallas as pl

_N_BLOCK = 2000


def _proj_kernel(x_ref, w_ref, o_ref):
    o_ref[...] = jnp.dot(x_ref[...], w_ref[...],
                         preferred_element_type=jnp.float32)


def _project(x, w):
    n, k = x.shape
    m = w.shape[1]
    grid = n // _N_BLOCK
    return pl.pallas_call(
        _proj_kernel,
        grid=(grid,),
        in_specs=[
            pl.BlockSpec((_N_BLOCK, k), lambda i: (i, 0)),
            pl.BlockSpec((k, m), lambda i: (0, 0)),
        ],
        out_specs=pl.BlockSpec((_N_BLOCK, m), lambda i: (i, 0)),
        out_shape=jax.ShapeDtypeStruct((n, m), jnp.float32),
    )(x, w)


def _edge_softmax(logits, dst, n):
    m = jax.ops.segment_max(logits, dst, num_segments=n)
    m = jnp.where(jnp.isfinite(m), m, 0.0)
    e = jnp.exp(logits - m[dst])
    s = jax.ops.segment_sum(e, dst, num_segments=n)
    return e / s[dst]


def _gat_conv(x, src, dst, p, heads, out_f):
    n = x.shape[0]
    wcat = jnp.concatenate([p["W"], p["Wres"]], axis=1)
    proj = _project(x, wcat)
    hf = heads * out_f
    feat = proj[:, :hf].reshape(n, heads, out_f)
    res = proj[:, hf:]
    el = (feat * p["al"][None]).sum(-1)
    er = (feat * p["ar"][None]).sum(-1)
    e = jax.nn.leaky_relu(el[src] + er[dst], negative_slope=0.2)
    a = _edge_softmax(e, dst, n)
    msg = feat[src] * a[:, :, None]
    rst = jax.ops.segment_sum(msg, dst, num_segments=n)
    rst = rst + res.reshape(n, heads, out_f)
    rst = rst + p["b"].reshape(1, heads, out_f)
    return rst


def kernel(x, edge_index_writes, edge_index_cites, params):
    edges = {"writes": edge_index_writes, "cites": edge_index_cites}
    rels = ("writes", "cites")
    n = x.shape[0]
    heads1, hid, out_dim = 4, 64, 128

    h1 = jnp.mean(
        jnp.stack([
            _gat_conv(x, edges[r][0], edges[r][1], params["l1"][r],
                      heads1, hid)
            for r in rels
        ]),
        axis=0,
    )
    h1 = jax.nn.elu(h1.reshape(n, heads1 * hid))
    h2 = jnp.mean(
        jnp.stack([
            _gat_conv(h1, edges[r][0], edges[r][1], params["l2"][r],
                      1, out_dim)
            for r in rels
        ]),
        axis=0,
    )
    return h2[:, 0, :]
